# Initial kernel scaffold; baseline (speedup 1.0000x reference)
import jax
import jax.numpy as jnp
from jax.experimental import pallas as pl


def kernel(signal_features, volatility, spread, W1, b1, W2, b2):
    raise NotImplementedError("write your pallas kernel here")


# speedup vs baseline: 28.6578x; 28.6578x over previous
import functools

import jax
import jax.numpy as jnp
from jax.experimental import pallas as pl
from jax.experimental.pallas import tpu as pltpu

_UNIVERSE_K = 512
_TRADE_K = 128


def _kth_largest_threshold(keys, k):
    def body(i, prefix):
        cand = prefix | (jnp.int32(1) << (30 - i))
        cnt = jnp.sum((keys >= cand).astype(jnp.int32), axis=1, keepdims=True)
        return jnp.where(cnt >= k, cand, prefix)

    prefix = jnp.zeros((keys.shape[0], 1), jnp.int32)
    return jax.lax.fori_loop(0, 31, body, prefix, unroll=True)


def _layer1_body(x_ref, w1_ref, b1_ref, h_ref):
    h = jnp.dot(x_ref[...], w1_ref[...], preferred_element_type=jnp.float32)
    h_ref[...] = jnp.maximum(h + b1_ref[...], 0.0)


def _select_body(h_ref, vol_ref, spr_ref, w2_ref, b2_ref, out_ref):
    logits = jnp.dot(h_ref[...], w2_ref[...], preferred_element_type=jnp.float32)
    s = jax.nn.sigmoid(logits + b2_ref[...])
    ls = s - 0.5

    ratio = vol_ref[...] / (spr_ref[...] + 1e-8)
    rbits = jax.lax.bitcast_convert_type(ratio, jnp.int32)
    t1 = _kth_largest_threshold(rbits, _UNIVERSE_K)
    universe = rbits >= t1

    abskey = jax.lax.bitcast_convert_type(jnp.abs(ls), jnp.int32) + 1
    key2 = jnp.where(universe, abskey, 0)
    t2 = _kth_largest_threshold(key2, _TRADE_K)
    mask = key2 >= t2

    selected = jnp.where(mask, ls, 0.0)
    denom = jnp.sum(jnp.abs(selected), axis=1, keepdims=True) + 1e-8
    out_ref[...] = selected / denom


def kernel(signal_features, volatility, spread, W1, b1, W2, b2):
    B, D_IN = signal_features.shape
    _, H = W1.shape
    N = W2.shape[1]

    BM1 = 256
    h = pl.pallas_call(
        _layer1_body,
        grid=(B // BM1,),
        in_specs=[
            pl.BlockSpec((BM1, D_IN), lambda i: (i, 0)),
            pl.BlockSpec((D_IN, H), lambda i: (0, 0)),
            pl.BlockSpec((1, H), lambda i: (0, 0)),
        ],
        out_specs=pl.BlockSpec((BM1, H), lambda i: (i, 0)),
        out_shape=jax.ShapeDtypeStruct((B, H), jnp.float32),
        compiler_params=pltpu.CompilerParams(
            dimension_semantics=("arbitrary",),
        ),
    )(signal_features, W1, b1.reshape(1, H))

    BM2 = 128
    action = pl.pallas_call(
        _select_body,
        grid=(B // BM2,),
        in_specs=[
            pl.BlockSpec((BM2, H), lambda i: (i, 0)),
            pl.BlockSpec((BM2, N), lambda i: (i, 0)),
            pl.BlockSpec((BM2, N), lambda i: (i, 0)),
            pl.BlockSpec((H, N), lambda i: (0, 0)),
            pl.BlockSpec((1, N), lambda i: (0, 0)),
        ],
        out_specs=pl.BlockSpec((BM2, N), lambda i: (i, 0)),
        out_shape=jax.ShapeDtypeStruct((B, N), jnp.float32),
        compiler_params=pltpu.CompilerParams(
            dimension_semantics=("arbitrary",),
        ),
    )(h, volatility, spread, W2, b2.reshape(1, N))
    return (action, jnp.zeros_like(action))


# speedup vs baseline: 36.0902x; 1.2593x over previous
import functools

import jax
import jax.numpy as jnp
from jax.experimental import pallas as pl
from jax.experimental.pallas import tpu as pltpu

_UNIVERSE_K = 512
_TRADE_K = 128


def _kth_largest_threshold(keys, k):
    def body(i, prefix):
        cand = prefix | (jnp.int32(1) << (30 - i))
        cnt = jnp.sum((keys >= cand).astype(jnp.int32), axis=1, keepdims=True)
        return jnp.where(cnt >= k, cand, prefix)

    prefix = jnp.zeros((keys.shape[0], 1), jnp.int32)
    return jax.lax.fori_loop(0, 31, body, prefix, unroll=True)


def _layer1_body(x_ref, w1_ref, b1_ref, vol_ref, spr_ref, h_ref, uni_ref):
    ratio = vol_ref[...] / (spr_ref[...] + 1e-8)
    rbits = jax.lax.bitcast_convert_type(ratio, jnp.int32)
    t1 = _kth_largest_threshold(rbits, _UNIVERSE_K)
    uni_ref[...] = (rbits >= t1).astype(jnp.int8)

    h = jnp.dot(x_ref[...], w1_ref[...], preferred_element_type=jnp.float32)
    h_ref[...] = jnp.maximum(h + b1_ref[...], 0.0)


def _select_body(h_ref, uni_ref, w2_ref, b2_ref, out_ref):
    logits = jnp.dot(h_ref[...], w2_ref[...], preferred_element_type=jnp.float32)
    s = jax.nn.sigmoid(logits + b2_ref[...])
    ls = s - 0.5

    universe = uni_ref[...] != 0

    abskey = jax.lax.bitcast_convert_type(jnp.abs(ls), jnp.int32) + 1
    key2 = jnp.where(universe, abskey, 0)
    t2 = _kth_largest_threshold(key2, _TRADE_K)
    mask = key2 >= t2

    selected = jnp.where(mask, ls, 0.0)
    denom = jnp.sum(jnp.abs(selected), axis=1, keepdims=True) + 1e-8
    out_ref[...] = selected / denom


def kernel(signal_features, volatility, spread, W1, b1, W2, b2):
    B, D_IN = signal_features.shape
    _, H = W1.shape
    N = W2.shape[1]

    BM1 = 256
    h, universe = pl.pallas_call(
        _layer1_body,
        grid=(B // BM1,),
        in_specs=[
            pl.BlockSpec((BM1, D_IN), lambda i: (i, 0)),
            pl.BlockSpec((D_IN, H), lambda i: (0, 0)),
            pl.BlockSpec((1, H), lambda i: (0, 0)),
            pl.BlockSpec((BM1, N), lambda i: (i, 0)),
            pl.BlockSpec((BM1, N), lambda i: (i, 0)),
        ],
        out_specs=[
            pl.BlockSpec((BM1, H), lambda i: (i, 0)),
            pl.BlockSpec((BM1, N), lambda i: (i, 0)),
        ],
        out_shape=[
            jax.ShapeDtypeStruct((B, H), jnp.float32),
            jax.ShapeDtypeStruct((B, N), jnp.int8),
        ],
        compiler_params=pltpu.CompilerParams(
            dimension_semantics=("arbitrary",),
        ),
    )(signal_features, W1, b1.reshape(1, H), volatility, spread)

    BM2 = 128
    action = pl.pallas_call(
        _select_body,
        grid=(B // BM2,),
        in_specs=[
            pl.BlockSpec((BM2, H), lambda i: (i, 0)),
            pl.BlockSpec((BM2, N), lambda i: (i, 0)),
            pl.BlockSpec((H, N), lambda i: (0, 0)),
            pl.BlockSpec((1, N), lambda i: (0, 0)),
        ],
        out_specs=pl.BlockSpec((BM2, N), lambda i: (i, 0)),
        out_shape=jax.ShapeDtypeStruct((B, N), jnp.float32),
        compiler_params=pltpu.CompilerParams(
            dimension_semantics=("arbitrary",),
        ),
    )(h, universe, W2, b2.reshape(1, N))
    return (action, jnp.zeros_like(action))
